# baseline (speedup 1.0000x reference)
import jax
import jax.numpy as jnp
from jax.experimental import pallas as pl

_DIM = 32
_LEVELS = 4
_K = 512
_BETA = 0.25
_BB = 8


def _top16(x):
    u = jax.lax.bitcast_convert_type(x, jnp.uint32)
    return jax.lax.bitcast_convert_type(u & jnp.uint32(0xFFFF0000), jnp.float32)


def _split3_bf16(x):
    hi = _top16(x)
    r1 = x - hi
    mid = _top16(r1)
    lo = r1 - mid
    return (hi.astype(jnp.bfloat16), mid.astype(jnp.bfloat16),
            lo.astype(jnp.bfloat16))


def _rvq_body(x_ref, emb2_ref, emb2bf_ref, ecat_ref,
              zq_ref, codes_ref, err_ref, hist_ref):
    first = pl.program_id(0) == 0
    err_acc = None
    hist_acc = None
    for b in range(x_ref.shape[0]):
        err_b, hist_b = _rvq_batch(b, x_ref, emb2_ref, emb2bf_ref, ecat_ref,
                                   zq_ref, codes_ref)
        err_acc = err_b if err_acc is None else err_acc + err_b
        hist_acc = hist_b if hist_acc is None else hist_acc + hist_b

    @pl.when(first)
    def _init():
        err_ref[...] = jnp.zeros_like(err_ref)
        hist_ref[...] = jnp.zeros_like(hist_ref)

    err_ref[...] += err_acc
    hist_ref[...] += hist_acc


def _rvq_batch(b, x_ref, emb2_ref, emb2bf_ref, ecat_ref, zq_ref, codes_ref):
    x = x_ref[b]
    T = x.shape[1]
    r = x
    sum_q = jnp.zeros_like(x)
    ones_col = jnp.ones((T, 1), dtype=jnp.bfloat16)
    iota16 = jax.lax.broadcasted_iota(jnp.int16, (_K, T), 0)
    r2 = jnp.sum(r * r, axis=0, keepdims=True)
    idx_rows = []
    err_rows = []
    hist_cols = []
    for l in range(_LEVELS):
        emb2 = emb2_ref[l]
        e2 = 0.25 * jnp.sum(emb2 * emb2, axis=1, keepdims=True)
        s2 = jax.lax.dot_general(
            emb2bf_ref[l], r.astype(jnp.bfloat16), (((1,), (0,)), ((), ())),
            preferred_element_type=jnp.float32)
        h = -(r2 - s2 + e2)
        idx = jnp.argmax(h, axis=0, keepdims=True)
        onehot = jnp.where(iota16 == idx.astype(jnp.int16),
                           jnp.bfloat16(1), jnp.bfloat16(0))
        qcat = jax.lax.dot_general(
            ecat_ref[l], onehot, (((1,), (0,)), ((), ())),
            preferred_element_type=jnp.float32)
        q = ((qcat[0 * _DIM:1 * _DIM]
              + qcat[1 * _DIM:2 * _DIM] * jnp.float32(2.0 ** -8))
             + qcat[2 * _DIM:3 * _DIM] * jnp.float32(2.0 ** -16))
        r = r - q
        sum_q = sum_q + q
        r2 = jnp.sum(r * r, axis=0, keepdims=True)
        idx_rows.append(idx)
        err_rows.append(r2)
        hist_cols.append(jax.lax.dot_general(
            onehot, ones_col, (((1,), (0,)), ((), ())),
            preferred_element_type=jnp.float32))
    zq_ref[b] = x + (sum_q - x)
    codes_ref[b] = jnp.concatenate(idx_rows, axis=0)
    err_blk = jnp.concatenate(err_rows, axis=0)
    hist_blk = jnp.concatenate(hist_cols, axis=1)
    return err_blk, hist_blk


def kernel(z, embeddings):
    B, C, H, W = z.shape
    T = H * W
    N = B * T
    zr = z.reshape(B, C, T)
    embt = jnp.transpose(embeddings, (0, 2, 1))
    hi, mid, lo = _split3_bf16(embt)
    ecat = jnp.concatenate(
        [hi, mid * jnp.bfloat16(2.0 ** 8), lo * jnp.bfloat16(2.0 ** 16)],
        axis=1)
    emb2 = embeddings * 2.0
    emb2bf = emb2.astype(jnp.bfloat16)

    out_shape = [
        jax.ShapeDtypeStruct((B, C, T), jnp.float32),
        jax.ShapeDtypeStruct((B, _LEVELS, T), jnp.int32),
        jax.ShapeDtypeStruct((_LEVELS, T), jnp.float32),
        jax.ShapeDtypeStruct((_K, _LEVELS), jnp.float32),
    ]
    full = lambda i: (0, 0, 0)
    zq, codes, err, hist = pl.pallas_call(
        _rvq_body,
        grid=(B // _BB,),
        in_specs=[
            pl.BlockSpec((_BB, C, T), lambda i: (i, 0, 0)),
            pl.BlockSpec((_LEVELS, _K, C), full),
            pl.BlockSpec((_LEVELS, _K, C), full),
            pl.BlockSpec((_LEVELS, 3 * C, _K), full),
        ],
        out_specs=[
            pl.BlockSpec((_BB, C, T), lambda i: (i, 0, 0)),
            pl.BlockSpec((_BB, _LEVELS, T), lambda i: (i, 0, 0)),
            pl.BlockSpec((_LEVELS, T), lambda i: (0, 0)),
            pl.BlockSpec((_K, _LEVELS), lambda i: (0, 0)),
        ],
        out_shape=out_shape,
    )(zr, emb2, emb2bf, ecat)

    z_q = zq.reshape(B, C, H, W)
    codes_out = codes.reshape(B, _LEVELS, H, W)
    mse = jnp.sum(err, axis=1) / (N * C)
    vq_loss = jnp.sum(mse + _BETA * mse)
    histt = hist.T
    probs = histt / (jnp.sum(histt, axis=1, keepdims=True) + 1e-09)
    entropy = -jnp.sum(probs * jnp.log(probs + 1e-09), axis=1)
    perplexity = jnp.mean(jnp.exp(entropy))
    return z_q, codes_out, vq_loss, perplexity


# speedup vs baseline: 1.0108x; 1.0108x over previous
import jax
import jax.numpy as jnp
from jax.experimental import pallas as pl

_DIM = 32
_LEVELS = 4
_K = 512
_BETA = 0.25
_BB = 8


def _top16(x):
    u = jax.lax.bitcast_convert_type(x, jnp.uint32)
    return jax.lax.bitcast_convert_type(u & jnp.uint32(0xFFFF0000), jnp.float32)


def _split3_bf16(x):
    hi = _top16(x)
    r1 = x - hi
    mid = _top16(r1)
    lo = r1 - mid
    return (hi.astype(jnp.bfloat16), mid.astype(jnp.bfloat16),
            lo.astype(jnp.bfloat16))


def _rvq_body(x_ref, emb2_ref, ecat_ref,
              zq_ref, codes_ref, err_ref, hist_ref):
    first = pl.program_id(0) == 0
    e2s = [0.25 * jnp.sum(emb2_ref[l] * emb2_ref[l], axis=1, keepdims=True)
           for l in range(_LEVELS)]
    err_acc = None
    hist_acc = None
    for b in range(x_ref.shape[0]):
        err_b, hist_b = _rvq_batch(b, x_ref, emb2_ref, e2s, ecat_ref,
                                   zq_ref, codes_ref)
        err_acc = err_b if err_acc is None else err_acc + err_b
        hist_acc = hist_b if hist_acc is None else hist_acc + hist_b

    @pl.when(first)
    def _init():
        err_ref[...] = jnp.zeros_like(err_ref)
        hist_ref[...] = jnp.zeros_like(hist_ref)

    err_ref[...] += err_acc
    hist_ref[...] += hist_acc


def _rvq_batch(b, x_ref, emb2_ref, e2s, ecat_ref, zq_ref, codes_ref):
    x = x_ref[b]
    T = x.shape[1]
    r = x
    sum_q = jnp.zeros_like(x)
    ones_col = jnp.ones((T, 1), dtype=jnp.bfloat16)
    iota16 = jax.lax.broadcasted_iota(jnp.int16, (_K, T), 0)
    r2 = jnp.sum(r * r, axis=0, keepdims=True)
    idx_rows = []
    err_rows = []
    hist_cols = []
    for l in range(_LEVELS):
        e2 = e2s[l]
        s2 = jax.lax.dot_general(
            emb2_ref[l], r, (((1,), (0,)), ((), ())),
            preferred_element_type=jnp.float32,
            precision=jax.lax.Precision.DEFAULT)
        h = -(r2 - s2 + e2)
        idx = jnp.argmax(h, axis=0, keepdims=True)
        onehot = jnp.where(iota16 == idx.astype(jnp.int16),
                           jnp.bfloat16(1), jnp.bfloat16(0))
        qcat = jax.lax.dot_general(
            ecat_ref[l], onehot, (((1,), (0,)), ((), ())),
            preferred_element_type=jnp.float32)
        q = ((qcat[0 * _DIM:1 * _DIM]
              + qcat[1 * _DIM:2 * _DIM] * jnp.float32(2.0 ** -8))
             + qcat[2 * _DIM:3 * _DIM] * jnp.float32(2.0 ** -16))
        r = r - q
        sum_q = sum_q + q
        r2 = jnp.sum(r * r, axis=0, keepdims=True)
        idx_rows.append(idx)
        err_rows.append(r2)
        hist_cols.append(jax.lax.dot_general(
            onehot, ones_col, (((1,), (0,)), ((), ())),
            preferred_element_type=jnp.float32))
    zq_ref[b] = x + (sum_q - x)
    codes_ref[b] = jnp.concatenate(idx_rows, axis=0)
    err_blk = jnp.concatenate(err_rows, axis=0)
    hist_blk = jnp.concatenate(hist_cols, axis=1)
    return err_blk, hist_blk


def kernel(z, embeddings):
    B, C, H, W = z.shape
    T = H * W
    N = B * T
    zr = z.reshape(B, C, T)
    embt = jnp.transpose(embeddings, (0, 2, 1))
    hi, mid, lo = _split3_bf16(embt)
    ecat = jnp.concatenate(
        [hi, mid * jnp.bfloat16(2.0 ** 8), lo * jnp.bfloat16(2.0 ** 16)],
        axis=1)
    emb2 = embeddings * 2.0

    out_shape = [
        jax.ShapeDtypeStruct((B, C, T), jnp.float32),
        jax.ShapeDtypeStruct((B, _LEVELS, T), jnp.int32),
        jax.ShapeDtypeStruct((_LEVELS, T), jnp.float32),
        jax.ShapeDtypeStruct((_K, _LEVELS), jnp.float32),
    ]
    full = lambda i: (0, 0, 0)
    zq, codes, err, hist = pl.pallas_call(
        _rvq_body,
        grid=(B // _BB,),
        in_specs=[
            pl.BlockSpec((_BB, C, T), lambda i: (i, 0, 0)),
            pl.BlockSpec((_LEVELS, _K, C), full),
            pl.BlockSpec((_LEVELS, 3 * C, _K), full),
        ],
        out_specs=[
            pl.BlockSpec((_BB, C, T), lambda i: (i, 0, 0)),
            pl.BlockSpec((_BB, _LEVELS, T), lambda i: (i, 0, 0)),
            pl.BlockSpec((_LEVELS, T), lambda i: (0, 0)),
            pl.BlockSpec((_K, _LEVELS), lambda i: (0, 0)),
        ],
        out_shape=out_shape,
    )(zr, emb2, ecat)

    z_q = zq.reshape(B, C, H, W)
    codes_out = codes.reshape(B, _LEVELS, H, W)
    mse = jnp.sum(err, axis=1) / (N * C)
    vq_loss = jnp.sum(mse + _BETA * mse)
    histt = hist.T
    probs = histt / (jnp.sum(histt, axis=1, keepdims=True) + 1e-09)
    entropy = -jnp.sum(probs * jnp.log(probs + 1e-09), axis=1)
    perplexity = jnp.mean(jnp.exp(entropy))
    return z_q, codes_out, vq_loss, perplexity
